# baseline (speedup 1.0000x reference)
import functools

import jax
import jax.numpy as jnp
from jax import lax
from jax.experimental import pallas as pl
from jax.experimental.pallas import tpu as pltpu
from jax.experimental.pallas import tpu_sc as plsc

N_USERS = 100000
M_ITEMS = 100000
DIM = 64
BATCH = 4096
K = 8
DECAY = 0.0001

NUM_WORKERS = 32
BPW = BATCH // NUM_WORKERS
LANES = 16
SLAB = 128
NEG_ROUNDS = 4
NEG_CHUNK = BPW * K // NEG_ROUNDS


@functools.cache
def _make_sc_kernel():
  mesh = plsc.VectorSubcoreMesh(core_axis_name="c", subcore_axis_name="s")

  @functools.partial(
      pl.kernel,
      mesh=mesh,
      compiler_params=pltpu.CompilerParams(needs_layout_passes=False,
                                           use_tc_tiling_on_sc=True),
      out_type=[
          jax.ShapeDtypeStruct((BATCH * K,), jnp.float32),
          jax.ShapeDtypeStruct((NUM_WORKERS, 128), jnp.float32),
      ],
      scratch_types=[
          pltpu.VMEM((BPW,), jnp.int32),
          pltpu.VMEM((BPW,), jnp.int32),
          pltpu.VMEM((K, BPW), jnp.int32),
          pltpu.VMEM((BPW,), jnp.int32),
          pltpu.VMEM((BPW,), jnp.int32),
          pltpu.VMEM((K, BPW), jnp.int32),
          pltpu.VMEM((BPW, SLAB), jnp.float32),
          pltpu.VMEM((BPW, SLAB), jnp.float32),
          pltpu.VMEM((NEG_CHUNK, SLAB), jnp.float32),
          pltpu.VMEM((NEG_CHUNK, SLAB), jnp.float32),
          pltpu.VMEM((BPW * K,), jnp.float32),
          pltpu.VMEM((128,), jnp.float32),
          pltpu.SemaphoreType.DMA,
          pltpu.SemaphoreType.DMA,
          pltpu.SemaphoreType.DMA,
      ],
  )
  def _sc_gather_score(users_hbm, pos_hbm, neg_hbm, utab_hbm, itab_hbm,
                       pn_hbm, norms_hbm,
                       uidx_v, pidx_v, nidx_v, uslab_v, pslab_v, nslab_v,
                       urows_v, prows_v, nrowsA_v, nrowsB_v,
                       pn_v, nrm_v, sem_up, semA, semB):
    wid = lax.axis_index("s") * 2 + lax.axis_index("c")
    base = wid * BPW

    pltpu.sync_copy(users_hbm.at[pl.ds(base, BPW)], uidx_v)
    pltpu.sync_copy(pos_hbm.at[pl.ds(base, BPW)], pidx_v)
    pltpu.sync_copy(neg_hbm.at[wid], nidx_v)

    for c in range(BPW // LANES):
      sl = pl.ds(c * LANES, LANES)
      uslab_v[sl] = lax.shift_right_logical(uidx_v[sl], 1)
      pslab_v[sl] = lax.shift_right_logical(pidx_v[sl], 1)
      for j in range(K):
        nslab_v[j, sl] = lax.shift_right_logical(nidx_v[j, sl], 1)

    cu = pltpu.async_copy(utab_hbm.at[uslab_v], urows_v, sem_up)
    cp = pltpu.async_copy(itab_hbm.at[pslab_v], prows_v, sem_up)
    nbufs = [nrowsA_v, nrowsB_v]
    nsems = [semA, semB]
    chunks_per_round = K // NEG_ROUNDS

    def fire_round(t):
      buf, sem = nbufs[t % 2], nsems[t % 2]
      cs = []
      for i in range(chunks_per_round):
        j = t * chunks_per_round + i
        cs.append(pltpu.async_copy(itab_hbm.at[nslab_v.at[j]],
                                   buf.at[pl.ds(i * BPW, BPW)], sem))
      return cs

    pend = {0: fire_round(0), 1: fire_round(1)}
    cu.wait()
    cp.wait()

    zero = jnp.zeros((LANES,), jnp.float32)
    iota = lax.iota(jnp.int32, LANES)
    s_u = zero
    s_p = zero
    s_n = zero
    groups_per_round = (BPW // LANES) // NEG_ROUNDS

    for t in range(NEG_ROUNDS):
      buf = nbufs[t % 2]
      for c in pend.pop(t):
        c.wait()

      for g in range(t * groups_per_round, (t + 1) * groups_per_round):
        bl = g * LANES + iota
        hu = (uidx_v[pl.ds(g * LANES, LANES)] & 1) * DIM
        hp = (pidx_v[pl.ds(g * LANES, LANES)] & 1) * DIM
        gsplat = iota * 0 + g
        hn = []
        rn = []
        for k in range(K):
          nvals = plsc.load_gather(nidx_v, [gsplat, k + iota * K])
          hn.append((nvals & 1) * DIM)
          rn.append(bl * K + k - t * NEG_CHUNK)

        def dim_step(d, carry, buf=buf, bl=bl, hu=hu, hp=hp, hn=hn, rn=rn):
          acc_p, acc_n, su, sp, sn = carry
          uv = plsc.load_gather(urows_v, [bl, hu + d])
          pv = plsc.load_gather(prows_v, [bl, hp + d])
          acc_p = acc_p + uv * pv
          su = su + uv * uv
          sp = sp + pv * pv
          new_n = []
          for k in range(K):
            nv = plsc.load_gather(buf, [rn[k], hn[k] + d])
            new_n.append(acc_n[k] + uv * nv)
            sn = sn + nv * nv
          return acc_p, tuple(new_n), su, sp, sn

        acc_p, acc_n, s_u, s_p, s_n = lax.fori_loop(
            0, DIM, dim_step, (zero, (zero,) * K, s_u, s_p, s_n))

        for k in range(K):
          plsc.store_scatter(pn_v, [bl * K + k], acc_p - acc_n[k])

      if t + 2 < NEG_ROUNDS:
        pend[t + 2] = fire_round(t + 2)

    nrm_v[pl.ds(0, LANES)] = s_u
    nrm_v[pl.ds(LANES, LANES)] = s_p
    nrm_v[pl.ds(2 * LANES, LANES)] = s_n * (1.0 / K)
    zpad = jnp.zeros((LANES,), jnp.float32)
    for c in range(3, 8):
      nrm_v[pl.ds(c * LANES, LANES)] = zpad

    pltpu.sync_copy(pn_v, pn_hbm.at[pl.ds(base * K, BPW * K)])
    pltpu.sync_copy(nrm_v, norms_hbm.at[wid])

  return _sc_gather_score


def _tc_loss_body(pn_ref, nrm_ref, mf_ref, emb_ref, tot_ref):
  x = -pn_ref[...]
  sp = jnp.maximum(x, 0.0) + jnp.log1p(jnp.exp(-jnp.abs(x)))
  mf = jnp.sum(sp) * (1.0 / (BATCH * K))
  reg = jnp.sum(nrm_ref[...]) * 0.5
  emb = (DECAY / BATCH) * reg
  one = jnp.ones((1, 1), jnp.float32)
  mf_ref[...] = mf * one
  emb_ref[...] = emb * one
  tot_ref[...] = (mf + emb) * one


def kernel(user_table, item_table, users, pos_items, neg_items):
  users_i = users.astype(jnp.int32)
  pos_i = pos_items.astype(jnp.int32)
  neg_i = neg_items.astype(jnp.int32).reshape(NUM_WORKERS, K, BPW)
  utab2 = user_table.reshape(N_USERS // 2, 2 * DIM)
  itab2 = item_table.reshape(M_ITEMS // 2, 2 * DIM)

  pn_flat, norms = _make_sc_kernel()(users_i, pos_i, neg_i, utab2, itab2)
  pos_neg = pn_flat.reshape(BATCH, K)

  mf, emb, tot = pl.pallas_call(
      _tc_loss_body,
      out_shape=[jax.ShapeDtypeStruct((1, 1), jnp.float32)] * 3,
  )(pn_flat.reshape(BATCH * K // 128, 128), norms)

  return (tot.reshape(()), mf.reshape(()), emb.reshape(()), pos_neg)
